# baseline (speedup 1.0000x reference)
import functools

import jax
import jax.numpy as jnp
from jax import lax
from jax.experimental import pallas as pl
from jax.experimental.pallas import tpu as pltpu
from jax.experimental.pallas import tpu_sc as plsc

N = 50000
E = 800000
H = 32
STEPS = 6
S2S_STEPS = 6

NCORE = 2
NSUB = 16
NTILE = NCORE * NSUB
CHUNK = 128
NB = 4
GRP = 7
NGROUP = 7
NBATCH = GRP * NGROUP
PER_TILE_ROWS = 3200
N_PAD = NSUB * PER_TILE_ROWS
TRASH = N
C_PER_TILE = NB * NBATCH
E_PAD = NTILE * C_PER_TILE * CHUNK
ROW_BLK = 5000


def _sc_segment_sum(out_nodes, idx5, zrows):
    mesh = plsc.VectorSubcoreMesh(core_axis_name="c", subcore_axis_name="s")

    @functools.partial(
        pl.kernel,
        mesh=mesh,
        compiler_params=pltpu.CompilerParams(use_tc_tiling_on_sc=False),
        out_type=jax.ShapeDtypeStruct((NCORE, N_PAD, H), jnp.float32),
        scratch_types=[
            pltpu.VMEM((2, GRP * NB, CHUNK), jnp.int32),
            pltpu.VMEM((NB, CHUNK, H), jnp.float32),
            pltpu.VMEM_SHARED((N_PAD, H), jnp.float32),
            pltpu.SemaphoreType.DMA,
            pltpu.SemaphoreType.DMA,
        ],
    )
    def sc_kernel(out_hbm, idx_hbm, z_hbm, g_hbm,
                  idx_v, rows_v, acc, semg, sems):
        c = lax.axis_index("c")
        s = lax.axis_index("s")
        w = s * NCORE + c

        pltpu.sync_copy(z_hbm, acc.at[pl.ds(s * PER_TILE_ROWS, PER_TILE_ROWS)])
        plsc.subcore_barrier()

        HB = NB // 2
        NH = GRP * NB // HB

        def fire_g(slot, h):
            for j in range(HB):
                pltpu.async_copy(
                    out_hbm.at[idx_v.at[0, 2 * h + j]], rows_v.at[slot + j],
                    semg)

        def wait_g_fire_s(slot, h):
            for j in range(HB):
                pltpu.make_async_copy(
                    out_hbm.at[idx_v.at[0, 2 * h + j]], rows_v.at[slot + j],
                    semg).wait()
                pltpu.async_copy(
                    rows_v.at[slot + j], acc.at[idx_v.at[1, 2 * h + j]],
                    sems, add=True)

        def drain_half():
            for j in range(HB):
                pltpu.make_async_copy(
                    rows_v.at[j], acc.at[idx_v.at[1, j]], sems).wait()

        def run_group():
            fire_g(0, 0)
            wait_g_fire_s(0, 0)
            fire_g(HB, 1)

            def pair_body(kk, _):
                h = 2 * kk + 1
                wait_g_fire_s(HB, h)
                drain_half()
                fire_g(0, h + 1)
                wait_g_fire_s(0, h + 1)
                drain_half()
                fire_g(HB, h + 2)
                return 0

            lax.fori_loop(0, (NH - 2) // 2, pair_body, 0)
            wait_g_fire_s(HB, NH - 1)
            drain_half()

        pltpu.sync_copy(idx_hbm.at[w, 0], idx_v)
        run_group()

        def group_body(k, _):
            drain_half()
            pltpu.sync_copy(idx_hbm.at[w, k], idx_v)
            run_group()
            return 0

        lax.fori_loop(1, NGROUP, group_body, 0)
        drain_half()
        plsc.subcore_barrier()

        r0 = s * PER_TILE_ROWS
        pltpu.sync_copy(acc.at[pl.ds(r0, PER_TILE_ROWS)],
                        g_hbm.at[c, pl.ds(r0, PER_TILE_ROWS)])

    return sc_kernel(out_nodes, idx5, zrows)


def _init_kernel(pos, seed2, W0a, w0b, b0):
    def body(pos_ref, seed_ref, W0a_ref, w0b_ref, b0_ref, o_ref):
        x = jnp.dot(pos_ref[...], W0a_ref[...],
                    preferred_element_type=jnp.float32)
        x = x + seed_ref[...] * w0b_ref[...] + b0_ref[...]
        o_ref[...] = jnp.maximum(x, 0.0)

    grid = N // ROW_BLK
    return pl.pallas_call(
        body,
        grid=(grid,),
        in_specs=[
            pl.BlockSpec((ROW_BLK, pos.shape[1]), lambda i: (i, 0)),
            pl.BlockSpec((ROW_BLK, 1), lambda i: (i, 0)),
            pl.BlockSpec(W0a.shape, lambda i: (0, 0)),
            pl.BlockSpec((1, H), lambda i: (0, 0)),
            pl.BlockSpec((1, H), lambda i: (0, 0)),
        ],
        out_specs=pl.BlockSpec((ROW_BLK, H), lambda i: (i, 0)),
        out_shape=jax.ShapeDtypeStruct((N, H), jnp.float32),
    )(pos, seed2, W0a, w0b, b0)


def _gru_kernel(g0, g1, h, Wm, bm, Wr, Wz, Wn, Ur, Uz, Un, br, bz, bn,
                hbr, hbz, hbn):
    def body(g0_ref, g1_ref, h_ref, Wm_ref, bm_ref, Wr_ref, Wz_ref, Wn_ref,
             Ur_ref, Uz_ref, Un_ref, br_ref, bz_ref, bn_ref,
             hbr_ref, hbz_ref, hbn_ref, o_ref):
        g = g0_ref[...] + g1_ref[...]
        m = jnp.maximum(
            jnp.dot(g, Wm_ref[...], preferred_element_type=jnp.float32)
            + bm_ref[...], 0.0)
        h = h_ref[...]
        i_r = jnp.dot(m, Wr_ref[...], preferred_element_type=jnp.float32) + br_ref[...]
        i_z = jnp.dot(m, Wz_ref[...], preferred_element_type=jnp.float32) + bz_ref[...]
        i_n = jnp.dot(m, Wn_ref[...], preferred_element_type=jnp.float32) + bn_ref[...]
        h_r = jnp.dot(h, Ur_ref[...], preferred_element_type=jnp.float32) + hbr_ref[...]
        h_z = jnp.dot(h, Uz_ref[...], preferred_element_type=jnp.float32) + hbz_ref[...]
        h_n = jnp.dot(h, Un_ref[...], preferred_element_type=jnp.float32) + hbn_ref[...]
        r = jax.nn.sigmoid(i_r + h_r)
        z = jax.nn.sigmoid(i_z + h_z)
        n = jnp.tanh(i_n + r * h_n)
        o_ref[...] = (1.0 - z) * n + z * h

    grid = N // ROW_BLK
    big = pl.BlockSpec((ROW_BLK, H), lambda i: (i, 0))
    sq = pl.BlockSpec((H, H), lambda i: (0, 0))
    vec = pl.BlockSpec((1, H), lambda i: (0, 0))
    return pl.pallas_call(
        body,
        grid=(grid,),
        in_specs=[big, big, big, sq, vec, sq, sq, sq, sq, sq, sq,
                  vec, vec, vec, vec, vec, vec],
        out_specs=big,
        out_shape=jax.ShapeDtypeStruct((N, H), jnp.float32),
    )(g0, g1, h, Wm, bm, Wr, Wz, Wn, Ur, Uz, Un, br, bz, bn, hbr, hbz, hbn)


def _s2s_kernel(out_nodes4, lstm_ws, W1a, W1b, b1, W2, b2):
    n_ws = len(lstm_ws)

    def body(*refs):
        out_ref = refs[0]
        ws = [r[...] for r in refs[1:1 + n_ws]]
        W1a_ref, W1b_ref, b1_ref, W2_ref, b2_ref, o_ref = refs[1 + n_ws:]

        r128a = lax.broadcasted_iota(jnp.int32, (128, 128), 0)
        c128a = lax.broadcasted_iota(jnp.int32, (128, 128), 1)
        S = (r128a // H == c128a // H).astype(jnp.float32)
        P = (lax.broadcasted_iota(jnp.int32, (128, H), 0) % H
             == lax.broadcasted_iota(jnp.int32, (128, H), 1)).astype(jnp.float32)
        PT = (lax.broadcasted_iota(jnp.int32, (H, 128), 1) % H
              == lax.broadcasted_iota(jnp.int32, (H, 128), 0)).astype(jnp.float32)

        k = 0
        A0 = ws[k:k + 4]; k += 4
        B0 = ws[k:k + 4]; k += 4
        U0 = ws[k:k + 4]; k += 4
        bb0 = ws[k:k + 4]; k += 4
        A1 = ws[k:k + 4]; k += 4
        U1 = ws[k:k + 4]; k += 4
        bb1 = ws[k:k + 4]; k += 4
        A2 = ws[k:k + 4]; k += 4
        U2 = ws[k:k + 4]; k += 4
        bb2 = ws[k:k + 4]; k += 4

        def dot(a, b):
            return jnp.dot(a, b, preferred_element_type=jnp.float32)

        def lstm(gates, h, c):
            i = jax.nn.sigmoid(gates[0])
            f = jax.nn.sigmoid(gates[1])
            gg = jnp.tanh(gates[2])
            o = jax.nn.sigmoid(gates[3])
            c_new = f * c + i * gg
            h_new = o * jnp.tanh(c_new)
            return h_new, c_new

        zero = jnp.zeros((1, H), jnp.float32)

        def step(_, carry):
            q, r, h0, c0, h1, c1, h2, c2 = carry
            g0 = [dot(q, A0[i]) + dot(r, B0[i]) + dot(h0, U0[i]) + bb0[i]
                  for i in range(4)]
            h0, c0 = lstm(g0, h0, c0)
            g1 = [dot(h0, A1[i]) + dot(h1, U1[i]) + bb1[i] for i in range(4)]
            h1, c1 = lstm(g1, h1, c1)
            g2 = [dot(h1, A2[i]) + dot(h2, U2[i]) + bb2[i] for i in range(4)]
            h2, c2 = lstm(g2, h2, c2)
            q = h2
            nodes = out_ref[...]
            qq = dot(q, PT)
            erep = dot(nodes * qq, S)
            p = jnp.exp(erep - jnp.max(erep))
            alpha = p * (jnp.float32(H) / jnp.sum(p))
            r128 = jnp.sum(alpha * nodes, axis=0, keepdims=True)
            r = dot(r128, P)
            return (q, r, h0, c0, h1, c1, h2, c2)

        carry = (zero, zero, zero, zero, zero, zero, zero, zero)
        q, r, *_ = lax.fori_loop(0, S2S_STEPS, step, carry)
        x = jnp.maximum(dot(q, W1a_ref[...]) + dot(r, W1b_ref[...])
                        + b1_ref[...], 0.0)
        o_ref[...] = dot(x, W2_ref[...]) + b2_ref[...]

    return pl.pallas_call(
        body,
        out_shape=jax.ShapeDtypeStruct((1, H), jnp.float32),
    )(out_nodes4, *lstm_ws, W1a, W1b, b1, W2, b2)


def kernel(pos_undirected, seed, W0, b0, Wm, bm, gru_W_ih, gru_W_hh,
           gru_b_ih, gru_b_hh, lstm_W_ih_0, lstm_W_hh_0, lstm_b_ih_0,
           lstm_b_hh_0, lstm_W_ih_1, lstm_W_hh_1, lstm_b_ih_1, lstm_b_hh_1,
           lstm_W_ih_2, lstm_W_hh_2, lstm_b_ih_2, lstm_b_hh_2, W1, b1, W2,
           b2, edge_index):
    f32 = jnp.float32
    POS = pos_undirected.shape[1]

    src = edge_index[0]
    dst = edge_index[1]
    pad = E_PAD - E
    src5 = jnp.concatenate([src, jnp.zeros((pad,), jnp.int32)]).reshape(
        NTILE, NGROUP, 1, GRP * NB, CHUNK)
    dst5 = jnp.concatenate([dst, jnp.full((pad,), TRASH, jnp.int32)]).reshape(
        NTILE, NGROUP, 1, GRP * NB, CHUNK)
    idx5 = jnp.concatenate([src5, dst5], axis=2)
    zrows = jnp.zeros((PER_TILE_ROWS, H), f32)

    W0a = W0[:POS]
    w0b = W0[POS:POS + 1]
    b0r = b0.reshape(1, H)
    bmr = bm.reshape(1, H)
    seed2 = seed.reshape(N, 1)

    def split3(W):
        return W[:, 0:H], W[:, H:2 * H], W[:, 2 * H:3 * H]

    Wr, Wz, Wn = split3(gru_W_ih)
    Ur, Uz, Un = split3(gru_W_hh)
    br, bz, bn = [x.reshape(1, H) for x in split3(gru_b_ih.reshape(1, -1))]
    hbr, hbz, hbn = [x.reshape(1, H) for x in split3(gru_b_hh.reshape(1, -1))]

    def split4(W):
        return [W[:, i * H:(i + 1) * H] for i in range(4)]

    lstm_ws = []
    lstm_ws += split4(lstm_W_ih_0[:H])
    lstm_ws += split4(lstm_W_ih_0[H:])
    lstm_ws += split4(lstm_W_hh_0)
    lstm_ws += [x.reshape(1, H)
                for x in split4((lstm_b_ih_0 + lstm_b_hh_0).reshape(1, -1))]
    lstm_ws += split4(lstm_W_ih_1)
    lstm_ws += split4(lstm_W_hh_1)
    lstm_ws += [x.reshape(1, H)
                for x in split4((lstm_b_ih_1 + lstm_b_hh_1).reshape(1, -1))]
    lstm_ws += split4(lstm_W_ih_2)
    lstm_ws += split4(lstm_W_hh_2)
    lstm_ws += [x.reshape(1, H)
                for x in split4((lstm_b_ih_2 + lstm_b_hh_2).reshape(1, -1))]

    W1a = W1[:H]
    W1b = W1[H:]
    b1r = b1.reshape(1, H)
    b2r = b2.reshape(1, -1)

    out = _init_kernel(pos_undirected, seed2, W0a, w0b, b0r)
    for _ in range(STEPS):
        g = _sc_segment_sum(out, idx5, zrows)
        out = _gru_kernel(g[0, :N], g[1, :N], out, Wm, bmr, Wr, Wz, Wn,
                          Ur, Uz, Un, br, bz, bn, hbr, hbz, hbn)
    return _s2s_kernel(out.reshape(N // 4, 4 * H), lstm_ws, W1a, W1b, b1r,
                       W2, b2r)


# speedup vs baseline: 1.0554x; 1.0554x over previous
import functools

import jax
import jax.numpy as jnp
from jax import lax
from jax.experimental import pallas as pl
from jax.experimental.pallas import tpu as pltpu
from jax.experimental.pallas import tpu_sc as plsc

N = 50000
E = 800000
H = 32
STEPS = 6
S2S_STEPS = 6

NCORE = 2
NSUB = 16
NTILE = NCORE * NSUB
CHUNK = 128
NB = 4
GRP = 7
NGROUP = 7
NBATCH = GRP * NGROUP
PER_TILE_ROWS = 3200
N_PAD = NSUB * PER_TILE_ROWS
TRASH = N
C_PER_TILE = NB * NBATCH
E_PAD = NTILE * C_PER_TILE * CHUNK
ROW_BLK = 5000


def _sc_segment_sum(out_nodes, idx5, zrows):
    mesh = plsc.VectorSubcoreMesh(core_axis_name="c", subcore_axis_name="s")

    @functools.partial(
        pl.kernel,
        mesh=mesh,
        compiler_params=pltpu.CompilerParams(use_tc_tiling_on_sc=False),
        out_type=jax.ShapeDtypeStruct((NCORE, N_PAD, H), jnp.float32),
        scratch_types=[
            pltpu.VMEM((2, GRP * NB, CHUNK), jnp.int32),
            pltpu.VMEM((NB, CHUNK, H), jnp.float32),
            pltpu.VMEM_SHARED((N_PAD, H), jnp.float32),
            pltpu.SemaphoreType.DMA,
            pltpu.SemaphoreType.DMA,
        ],
    )
    def sc_kernel(out_hbm, idx_hbm, z_hbm, g_hbm,
                  idx_v, rows_v, acc, semg, sems):
        c = lax.axis_index("c")
        s = lax.axis_index("s")
        w = s * NCORE + c

        pltpu.sync_copy(z_hbm, acc.at[pl.ds(s * PER_TILE_ROWS, PER_TILE_ROWS)])
        plsc.subcore_barrier()

        def drain_batch():
            for b in range(NB):
                pltpu.make_async_copy(
                    rows_v.at[b], acc.at[idx_v.at[1, b]], sems).wait()

        def do_batch(m, drain_prev):
            if drain_prev:
                drain_batch()
            gathers = []
            for b in range(NB):
                gathers.append(pltpu.async_copy(
                    out_hbm.at[idx_v.at[0, m * NB + b]], rows_v.at[b], semg))
            for b in range(NB):
                gathers[b].wait()
                pltpu.async_copy(rows_v.at[b], acc.at[idx_v.at[1, m * NB + b]],
                                 sems, add=True)

        def batch_body(m, _):
            do_batch(m, True)
            return 0

        pltpu.sync_copy(idx_hbm.at[w, 0], idx_v)
        do_batch(0, False)
        lax.fori_loop(1, GRP, batch_body, 0)

        def group_body(k, _):
            drain_batch()
            pltpu.sync_copy(idx_hbm.at[w, k], idx_v)
            do_batch(0, False)
            lax.fori_loop(1, GRP, batch_body, 0)
            return 0

        lax.fori_loop(1, NGROUP, group_body, 0)
        drain_batch()
        plsc.subcore_barrier()

        r0 = s * PER_TILE_ROWS
        pltpu.sync_copy(acc.at[pl.ds(r0, PER_TILE_ROWS)],
                        g_hbm.at[c, pl.ds(r0, PER_TILE_ROWS)])

    return sc_kernel(out_nodes, idx5, zrows)


def _init_kernel(pos, seed2, W0a, w0b, b0):
    def body(pos_ref, seed_ref, W0a_ref, w0b_ref, b0_ref, o_ref):
        x = jnp.dot(pos_ref[...], W0a_ref[...],
                    preferred_element_type=jnp.float32)
        x = x + seed_ref[...] * w0b_ref[...] + b0_ref[...]
        o_ref[...] = jnp.maximum(x, 0.0)

    grid = N // ROW_BLK
    return pl.pallas_call(
        body,
        grid=(grid,),
        in_specs=[
            pl.BlockSpec((ROW_BLK, pos.shape[1]), lambda i: (i, 0)),
            pl.BlockSpec((ROW_BLK, 1), lambda i: (i, 0)),
            pl.BlockSpec(W0a.shape, lambda i: (0, 0)),
            pl.BlockSpec((1, H), lambda i: (0, 0)),
            pl.BlockSpec((1, H), lambda i: (0, 0)),
        ],
        out_specs=pl.BlockSpec((ROW_BLK, H), lambda i: (i, 0)),
        out_shape=jax.ShapeDtypeStruct((N, H), jnp.float32),
    )(pos, seed2, W0a, w0b, b0)


def _gru_kernel(g0, g1, h, Wm, bm, Wr, Wz, Wn, Ur, Uz, Un, br, bz, bn,
                hbr, hbz, hbn):
    def body(g0_ref, g1_ref, h_ref, Wm_ref, bm_ref, Wr_ref, Wz_ref, Wn_ref,
             Ur_ref, Uz_ref, Un_ref, br_ref, bz_ref, bn_ref,
             hbr_ref, hbz_ref, hbn_ref, o_ref):
        g = g0_ref[...] + g1_ref[...]
        m = jnp.maximum(
            jnp.dot(g, Wm_ref[...], preferred_element_type=jnp.float32)
            + bm_ref[...], 0.0)
        h = h_ref[...]
        i_r = jnp.dot(m, Wr_ref[...], preferred_element_type=jnp.float32) + br_ref[...]
        i_z = jnp.dot(m, Wz_ref[...], preferred_element_type=jnp.float32) + bz_ref[...]
        i_n = jnp.dot(m, Wn_ref[...], preferred_element_type=jnp.float32) + bn_ref[...]
        h_r = jnp.dot(h, Ur_ref[...], preferred_element_type=jnp.float32) + hbr_ref[...]
        h_z = jnp.dot(h, Uz_ref[...], preferred_element_type=jnp.float32) + hbz_ref[...]
        h_n = jnp.dot(h, Un_ref[...], preferred_element_type=jnp.float32) + hbn_ref[...]
        r = jax.nn.sigmoid(i_r + h_r)
        z = jax.nn.sigmoid(i_z + h_z)
        n = jnp.tanh(i_n + r * h_n)
        o_ref[...] = (1.0 - z) * n + z * h

    grid = N // ROW_BLK
    big = pl.BlockSpec((ROW_BLK, H), lambda i: (i, 0))
    sq = pl.BlockSpec((H, H), lambda i: (0, 0))
    vec = pl.BlockSpec((1, H), lambda i: (0, 0))
    return pl.pallas_call(
        body,
        grid=(grid,),
        in_specs=[big, big, big, sq, vec, sq, sq, sq, sq, sq, sq,
                  vec, vec, vec, vec, vec, vec],
        out_specs=big,
        out_shape=jax.ShapeDtypeStruct((N, H), jnp.float32),
    )(g0, g1, h, Wm, bm, Wr, Wz, Wn, Ur, Uz, Un, br, bz, bn, hbr, hbz, hbn)


def _s2s_kernel(out_nodes4, lstm_ws, W1a, W1b, b1, W2, b2):
    n_ws = len(lstm_ws)

    def body(*refs):
        out_ref = refs[0]
        ws = [r[...] for r in refs[1:1 + n_ws]]
        W1a_ref, W1b_ref, b1_ref, W2_ref, b2_ref, o_ref = refs[1 + n_ws:]

        r128a = lax.broadcasted_iota(jnp.int32, (128, 128), 0)
        c128a = lax.broadcasted_iota(jnp.int32, (128, 128), 1)
        S = (r128a // H == c128a // H).astype(jnp.float32)
        P = (lax.broadcasted_iota(jnp.int32, (128, H), 0) % H
             == lax.broadcasted_iota(jnp.int32, (128, H), 1)).astype(jnp.float32)
        PT = (lax.broadcasted_iota(jnp.int32, (H, 128), 1) % H
              == lax.broadcasted_iota(jnp.int32, (H, 128), 0)).astype(jnp.float32)

        k = 0
        A0 = ws[k:k + 4]; k += 4
        B0 = ws[k:k + 4]; k += 4
        U0 = ws[k:k + 4]; k += 4
        bb0 = ws[k:k + 4]; k += 4
        A1 = ws[k:k + 4]; k += 4
        U1 = ws[k:k + 4]; k += 4
        bb1 = ws[k:k + 4]; k += 4
        A2 = ws[k:k + 4]; k += 4
        U2 = ws[k:k + 4]; k += 4
        bb2 = ws[k:k + 4]; k += 4

        def dot(a, b):
            return jnp.dot(a, b, preferred_element_type=jnp.float32)

        def lstm(gates, h, c):
            i = jax.nn.sigmoid(gates[0])
            f = jax.nn.sigmoid(gates[1])
            gg = jnp.tanh(gates[2])
            o = jax.nn.sigmoid(gates[3])
            c_new = f * c + i * gg
            h_new = o * jnp.tanh(c_new)
            return h_new, c_new

        zero = jnp.zeros((1, H), jnp.float32)

        def step(_, carry):
            q, r, h0, c0, h1, c1, h2, c2 = carry
            g0 = [dot(q, A0[i]) + dot(r, B0[i]) + dot(h0, U0[i]) + bb0[i]
                  for i in range(4)]
            h0, c0 = lstm(g0, h0, c0)
            g1 = [dot(h0, A1[i]) + dot(h1, U1[i]) + bb1[i] for i in range(4)]
            h1, c1 = lstm(g1, h1, c1)
            g2 = [dot(h1, A2[i]) + dot(h2, U2[i]) + bb2[i] for i in range(4)]
            h2, c2 = lstm(g2, h2, c2)
            q = h2
            nodes = out_ref[...]
            qq = dot(q, PT)
            erep = dot(nodes * qq, S)
            p = jnp.exp(erep - jnp.max(erep))
            alpha = p * (jnp.float32(H) / jnp.sum(p))
            r128 = jnp.sum(alpha * nodes, axis=0, keepdims=True)
            r = dot(r128, P)
            return (q, r, h0, c0, h1, c1, h2, c2)

        carry = (zero, zero, zero, zero, zero, zero, zero, zero)
        q, r, *_ = lax.fori_loop(0, S2S_STEPS, step, carry)
        x = jnp.maximum(dot(q, W1a_ref[...]) + dot(r, W1b_ref[...])
                        + b1_ref[...], 0.0)
        o_ref[...] = dot(x, W2_ref[...]) + b2_ref[...]

    return pl.pallas_call(
        body,
        out_shape=jax.ShapeDtypeStruct((1, H), jnp.float32),
    )(out_nodes4, *lstm_ws, W1a, W1b, b1, W2, b2)


def kernel(pos_undirected, seed, W0, b0, Wm, bm, gru_W_ih, gru_W_hh,
           gru_b_ih, gru_b_hh, lstm_W_ih_0, lstm_W_hh_0, lstm_b_ih_0,
           lstm_b_hh_0, lstm_W_ih_1, lstm_W_hh_1, lstm_b_ih_1, lstm_b_hh_1,
           lstm_W_ih_2, lstm_W_hh_2, lstm_b_ih_2, lstm_b_hh_2, W1, b1, W2,
           b2, edge_index):
    f32 = jnp.float32
    POS = pos_undirected.shape[1]

    src = edge_index[0]
    dst = edge_index[1]
    pad = E_PAD - E
    src5 = jnp.concatenate([src, jnp.zeros((pad,), jnp.int32)]).reshape(
        NTILE, NGROUP, 1, GRP * NB, CHUNK)
    dst5 = jnp.concatenate([dst, jnp.full((pad,), TRASH, jnp.int32)]).reshape(
        NTILE, NGROUP, 1, GRP * NB, CHUNK)
    idx5 = jnp.concatenate([src5, dst5], axis=2)
    zrows = jnp.zeros((PER_TILE_ROWS, H), f32)

    W0a = W0[:POS]
    w0b = W0[POS:POS + 1]
    b0r = b0.reshape(1, H)
    bmr = bm.reshape(1, H)
    seed2 = seed.reshape(N, 1)

    def split3(W):
        return W[:, 0:H], W[:, H:2 * H], W[:, 2 * H:3 * H]

    Wr, Wz, Wn = split3(gru_W_ih)
    Ur, Uz, Un = split3(gru_W_hh)
    br, bz, bn = [x.reshape(1, H) for x in split3(gru_b_ih.reshape(1, -1))]
    hbr, hbz, hbn = [x.reshape(1, H) for x in split3(gru_b_hh.reshape(1, -1))]

    def split4(W):
        return [W[:, i * H:(i + 1) * H] for i in range(4)]

    lstm_ws = []
    lstm_ws += split4(lstm_W_ih_0[:H])
    lstm_ws += split4(lstm_W_ih_0[H:])
    lstm_ws += split4(lstm_W_hh_0)
    lstm_ws += [x.reshape(1, H)
                for x in split4((lstm_b_ih_0 + lstm_b_hh_0).reshape(1, -1))]
    lstm_ws += split4(lstm_W_ih_1)
    lstm_ws += split4(lstm_W_hh_1)
    lstm_ws += [x.reshape(1, H)
                for x in split4((lstm_b_ih_1 + lstm_b_hh_1).reshape(1, -1))]
    lstm_ws += split4(lstm_W_ih_2)
    lstm_ws += split4(lstm_W_hh_2)
    lstm_ws += [x.reshape(1, H)
                for x in split4((lstm_b_ih_2 + lstm_b_hh_2).reshape(1, -1))]

    W1a = W1[:H]
    W1b = W1[H:]
    b1r = b1.reshape(1, H)
    b2r = b2.reshape(1, -1)

    out = _init_kernel(pos_undirected, seed2, W0a, w0b, b0r)
    for _ in range(STEPS):
        g = _sc_segment_sum(out, idx5, zrows)
        out = _gru_kernel(g[0, :N], g[1, :N], out, Wm, bmr, Wr, Wz, Wn,
                          Ur, Uz, Un, br, bz, bn, hbr, hbz, hbn)
    return _s2s_kernel(out.reshape(N // 4, 4 * H), lstm_ws, W1a, W1b, b1r,
                       W2, b2r)
